# Initial kernel scaffold; baseline (speedup 1.0000x reference)
import jax
import jax.numpy as jnp
from jax.experimental import pallas as pl


def kernel(particle_coords, particle_values, xyz_min, xyz_max):
    raise NotImplementedError("write your pallas kernel here")


# speedup vs baseline: 2.3381x; 2.3381x over previous
import functools
import jax
import jax.numpy as jnp
from jax import lax
from jax.experimental import pallas as pl
from jax.experimental.pallas import tpu as pltpu
from jax.experimental.pallas import tpu_sc as plsc

D = H = W = 128
PLANE = H * W
N_PART = 2097152
N_CH = 4
PLANES_PER_REGION = 16
REGION_CELLS = PLANES_PER_REGION * PLANE
PAD_ROWS = 2048
ACC_ROWS = REGION_CELLS + PAD_ROWS
NUM_PASSES = 4
CHUNK = 128
NC, NS, L = 2, 16, 16
PER_TEC = N_PART // NS
N_CHUNKS = PER_TEC // CHUNK
ZROWS = ACC_ROWS // NS
NORM_CHUNK = 2048
CORNERS = [(di, dj, dk) for di in (0, 1) for dj in (0, 1) for dk in (0, 1)]


def _make_sc_kernel():
    mesh = plsc.VectorSubcoreMesh(core_axis_name="c", subcore_axis_name="s")
    scratch = []
    scratch += [pltpu.VMEM_SHARED((ACC_ROWS,), jnp.float32) for _ in range(5)]
    scratch += [
        pltpu.VMEM((CHUNK * 3,), jnp.float32),
        pltpu.VMEM((CHUNK * 4,), jnp.float32),
        pltpu.VMEM((96,), jnp.float32),
    ]
    scratch += [pltpu.VMEM((CHUNK,), jnp.int32) for _ in range(8)]
    scratch += [pltpu.VMEM((CHUNK,), jnp.float32) for _ in range(40)]
    scratch += [pltpu.VMEM((NORM_CHUNK,), jnp.float32) for _ in range(5)]
    scratch += [pltpu.VMEM((NORM_CHUNK,), jnp.float32) for _ in range(4)]
    scratch += [pltpu.SemaphoreType.DMA]

    @functools.partial(
        pl.kernel,
        mesh=mesh,
        out_type=jax.ShapeDtypeStruct((N_CH, D * H * W), jnp.float32),
        scratch_types=scratch,
        compiler_params=pltpu.CompilerParams(needs_layout_passes=False),
    )
    def sc_kernel(coords, values, consts, zeros_hbm, out, *refs):
        accs = list(refs[0:5])
        cbuf, vbuf, constv = refs[5], refs[6], refs[7]
        idx_bufs = list(refs[8:16])
        pay_bufs = [list(refs[16 + 5 * cn:16 + 5 * cn + 5]) for cn in range(8)]
        nbufs = list(refs[56:61])
        out_bufs = list(refs[61:65])
        sem = refs[65]

        c = lax.axis_index("c")
        s = lax.axis_index("s")
        wid = c * NS + s

        pltpu.sync_copy(consts, constv)
        lanes = lax.iota(jnp.int32, L)

        def do_pass(p, _):
            r0_plane = (2 * p + c) * PLANES_PER_REGION
            gcell0 = r0_plane * PLANE

            for a in range(5):
                pltpu.sync_copy(zeros_hbm, accs[a].at[pl.ds(s * ZROWS, ZROWS)])
            plsc.subcore_barrier()

            sxv = constv[pl.ds(0, L)]
            syv = constv[pl.ds(16, L)]
            szv = constv[pl.ds(32, L)]
            oxv = constv[pl.ds(48, L)]
            oyv = constv[pl.ds(64, L)]
            ozv = constv[pl.ds(80, L)]
            zero_f = jnp.zeros((L,), jnp.float32)
            hi_f = jnp.full((L,), 127.0, jnp.float32)
            hi_i = jnp.full((L,), 126, jnp.int32)
            one_f = jnp.full((L,), 1.0, jnp.float32)
            padbase = jnp.full((L,), REGION_CELLS, jnp.int32)
            r0i = jnp.full((L,), PLANES_PER_REGION, jnp.int32) * (2 * p + c)

            def chunk_body(ci, _):
                pbase = s * PER_TEC + ci * CHUNK
                pltpu.sync_copy(coords.at[pl.ds(pbase * 3, CHUNK * 3)], cbuf)
                pltpu.sync_copy(values.at[pl.ds(pbase * 4, CHUNK * 4)], vbuf)
                for v in range(CHUNK // L):
                    q = lanes + (L * v)
                    q3 = q * 3
                    q4 = q << 2
                    x = plsc.load_gather(cbuf, [q3])
                    y = plsc.load_gather(cbuf, [q3 + 1])
                    z = plsc.load_gather(cbuf, [q3 + 2])
                    va = plsc.load_gather(vbuf, [q4])
                    vb = plsc.load_gather(vbuf, [q4 + 1])
                    vc = plsc.load_gather(vbuf, [q4 + 2])
                    vd = plsc.load_gather(vbuf, [q4 + 3])

                    fi = jnp.minimum(jnp.maximum(x * sxv + oxv, zero_f), hi_f)
                    fj = jnp.minimum(jnp.maximum(y * syv + oyv, zero_f), hi_f)
                    fk = jnp.minimum(jnp.maximum(z * szv + ozv, zero_f), hi_f)
                    bi = jnp.minimum(fi.astype(jnp.int32), hi_i)
                    bj = jnp.minimum(fj.astype(jnp.int32), hi_i)
                    bk = jnp.minimum(fk.astype(jnp.int32), hi_i)
                    f0 = fi - bi.astype(jnp.float32)
                    f1 = fj - bj.astype(jnp.float32)
                    f2 = fk - bk.astype(jnp.float32)
                    g0 = one_f - f0
                    g1 = one_f - f1
                    g2 = one_f - f2
                    reli = bi - r0i

                    for cn, (di, dj, dk) in enumerate(CORNERS):
                        wi = f0 if di else g0
                        wj = f1 if dj else g1
                        wk = f2 if dk else g2
                        wgt = wi * (wj * wk)
                        rel = reli + di
                        valid = (rel >= 0) & (rel < PLANES_PER_REGION)
                        lidx = (rel << 14) + ((bj + dj) << 7) + (bk + dk)
                        lidx = jnp.where(valid, lidx, padbase + q)
                        wm = jnp.where(valid, wgt, zero_f)
                        sl = pl.ds(L * v, L)
                        idx_bufs[cn][sl] = lidx
                        pay_bufs[cn][0][sl] = wm * va
                        pay_bufs[cn][1][sl] = wm * vb
                        pay_bufs[cn][2][sl] = wm * vc
                        pay_bufs[cn][3][sl] = wm * vd
                        pay_bufs[cn][4][sl] = wm
                for cn in range(8):
                    for a in range(5):
                        pltpu.sync_copy(
                            pay_bufs[cn][a],
                            accs[a].at[idx_bufs[cn]],
                            add=True,
                        )
                return 0

            lax.fori_loop(0, N_CHUNKS, chunk_body, 0)
            plsc.subcore_barrier()

            eps = jnp.full((L,), 1e-8, jnp.float32)

            def norm_body(j, _):
                cell0 = s * PLANE + j * NORM_CHUNK
                for a in range(5):
                    pltpu.sync_copy(accs[a].at[pl.ds(cell0, NORM_CHUNK)], nbufs[a])

                def vec_body(vv, _):
                    sl = pl.ds(vv * L, L)
                    wsum = nbufs[4][sl]
                    inv = one_f / (wsum + eps)
                    for ch in range(N_CH):
                        out_bufs[ch][sl] = nbufs[ch][sl] * inv
                    return 0

                lax.fori_loop(0, NORM_CHUNK // L, vec_body, 0)
                for ch in range(N_CH):
                    pltpu.sync_copy(
                        out_bufs[ch], out.at[ch, pl.ds(gcell0 + cell0, NORM_CHUNK)]
                    )
                return 0

            lax.fori_loop(0, PLANE // NORM_CHUNK, norm_body, 0)
            plsc.subcore_barrier()
            return 0

        lax.fori_loop(0, NUM_PASSES, do_pass, 0)

    return sc_kernel


_SC_KERNEL = _make_sc_kernel()


def kernel(particle_coords, particle_values, xyz_min, xyz_max):
    scale = 127.0 / (xyz_max - xyz_min)
    off = -xyz_min * scale
    consts = jnp.repeat(jnp.concatenate([scale, off]), 16).astype(jnp.float32)
    zeros_hbm = jnp.zeros((ZROWS,), jnp.float32)
    out = _SC_KERNEL(
        particle_coords.reshape(-1), particle_values.reshape(-1), consts, zeros_hbm
    )
    return out.reshape(N_CH, D, H, W)


# speedup vs baseline: 3.7312x; 1.5959x over previous
import functools
import jax
import jax.numpy as jnp
from jax import lax
from jax.experimental import pallas as pl
from jax.experimental.pallas import tpu as pltpu
from jax.experimental.pallas import tpu_sc as plsc

D = H = W = 128
PLANE = H * W
N_PART = 2097152
N_CH = 4
PLANES_PER_REGION = 16
REGION_CELLS = PLANES_PER_REGION * PLANE
PAD_ROWS = 2048
ACC_ROWS = REGION_CELLS + PAD_ROWS
NUM_PASSES = 4
CHUNK = 128
NC, NS, L = 2, 16, 16
PER_TEC = N_PART // NS
N_CHUNKS = PER_TEC // CHUNK
ZROWS = ACC_ROWS // NS
NORM_CHUNK = 2048
CORNERS = [(di, dj, dk) for di in (0, 1) for dj in (0, 1) for dk in (0, 1)]


def _make_sc_kernel():
    mesh = plsc.VectorSubcoreMesh(core_axis_name="c", subcore_axis_name="s")
    scratch = []
    scratch += [pltpu.VMEM_SHARED((ACC_ROWS,), jnp.float32) for _ in range(5)]
    scratch += [
        pltpu.VMEM((CHUNK * 3,), jnp.float32),
        pltpu.VMEM((CHUNK * 4,), jnp.float32),
        pltpu.VMEM((96,), jnp.float32),
    ]
    scratch += [pltpu.VMEM((CHUNK,), jnp.int32) for _ in range(8)]
    scratch += [pltpu.VMEM((CHUNK,), jnp.float32) for _ in range(40)]
    scratch += [pltpu.VMEM((NORM_CHUNK,), jnp.float32) for _ in range(5)]
    scratch += [pltpu.VMEM((NORM_CHUNK,), jnp.float32) for _ in range(4)]
    scratch += [pltpu.SemaphoreType.DMA]

    @functools.partial(
        pl.kernel,
        mesh=mesh,
        out_type=jax.ShapeDtypeStruct((N_CH, D * H * W), jnp.float32),
        scratch_types=scratch,
        compiler_params=pltpu.CompilerParams(needs_layout_passes=False),
    )
    def sc_kernel(coords, values, consts, zeros_hbm, out, *refs):
        accs = list(refs[0:5])
        cbuf, vbuf, constv = refs[5], refs[6], refs[7]
        idx_bufs = list(refs[8:16])
        pay_bufs = [list(refs[16 + 5 * cn:16 + 5 * cn + 5]) for cn in range(8)]
        nbufs = list(refs[56:61])
        out_bufs = list(refs[61:65])
        sem = refs[65]

        c = lax.axis_index("c")
        s = lax.axis_index("s")
        wid = c * NS + s

        pltpu.sync_copy(consts, constv)
        lanes = lax.iota(jnp.int32, L)

        def do_pass(p, _):
            r0_plane = (2 * p + c) * PLANES_PER_REGION
            gcell0 = r0_plane * PLANE

            for a in range(5):
                pltpu.sync_copy(zeros_hbm, accs[a].at[pl.ds(s * ZROWS, ZROWS)])
            plsc.subcore_barrier()

            sxv = constv[pl.ds(0, L)]
            syv = constv[pl.ds(16, L)]
            szv = constv[pl.ds(32, L)]
            oxv = constv[pl.ds(48, L)]
            oyv = constv[pl.ds(64, L)]
            ozv = constv[pl.ds(80, L)]
            zero_f = jnp.zeros((L,), jnp.float32)
            hi_f = jnp.full((L,), 127.0, jnp.float32)
            hi_i = jnp.full((L,), 126, jnp.int32)
            one_f = jnp.full((L,), 1.0, jnp.float32)
            padbase = jnp.full((L,), REGION_CELLS, jnp.int32)
            r0i = jnp.full((L,), PLANES_PER_REGION, jnp.int32) * (2 * p + c)

            def chunk_body(ci, _):
                pbase = s * PER_TEC + ci * CHUNK
                pltpu.sync_copy(coords.at[pl.ds(pbase * 3, CHUNK * 3)], cbuf)
                pltpu.sync_copy(values.at[pl.ds(pbase * 4, CHUNK * 4)], vbuf)
                for v in range(CHUNK // L):
                    q = lanes + (L * v)
                    q3 = q * 3
                    q4 = q << 2
                    x = plsc.load_gather(cbuf, [q3])
                    y = plsc.load_gather(cbuf, [q3 + 1])
                    z = plsc.load_gather(cbuf, [q3 + 2])
                    va = plsc.load_gather(vbuf, [q4])
                    vb = plsc.load_gather(vbuf, [q4 + 1])
                    vc = plsc.load_gather(vbuf, [q4 + 2])
                    vd = plsc.load_gather(vbuf, [q4 + 3])

                    fi = jnp.minimum(jnp.maximum(x * sxv + oxv, zero_f), hi_f)
                    fj = jnp.minimum(jnp.maximum(y * syv + oyv, zero_f), hi_f)
                    fk = jnp.minimum(jnp.maximum(z * szv + ozv, zero_f), hi_f)
                    bi = jnp.minimum(fi.astype(jnp.int32), hi_i)
                    bj = jnp.minimum(fj.astype(jnp.int32), hi_i)
                    bk = jnp.minimum(fk.astype(jnp.int32), hi_i)
                    f0 = fi - bi.astype(jnp.float32)
                    f1 = fj - bj.astype(jnp.float32)
                    f2 = fk - bk.astype(jnp.float32)
                    g0 = one_f - f0
                    g1 = one_f - f1
                    g2 = one_f - f2
                    reli = bi - r0i

                    for cn, (di, dj, dk) in enumerate(CORNERS):
                        wi = f0 if di else g0
                        wj = f1 if dj else g1
                        wk = f2 if dk else g2
                        wgt = wi * (wj * wk)
                        rel = reli + di
                        valid = (rel >= 0) & (rel < PLANES_PER_REGION)
                        lidx = (rel << 14) + ((bj + dj) << 7) + (bk + dk)
                        lidx = jnp.where(valid, lidx, padbase + q)
                        wm = jnp.where(valid, wgt, zero_f)
                        sl = pl.ds(L * v, L)
                        idx_bufs[cn][sl] = lidx
                        pay_bufs[cn][0][sl] = wm * va
                        pay_bufs[cn][1][sl] = wm * vb
                        pay_bufs[cn][2][sl] = wm * vc
                        pay_bufs[cn][3][sl] = wm * vd
                        pay_bufs[cn][4][sl] = wm
                copies = []
                for cn in range(8):
                    for a in range(5):
                        copies.append(
                            pltpu.async_copy(
                                pay_bufs[cn][a],
                                accs[a].at[idx_bufs[cn]],
                                sem,
                                add=True,
                            )
                        )
                for cp in copies:
                    cp.wait()
                return 0

            lax.fori_loop(0, N_CHUNKS, chunk_body, 0)
            plsc.subcore_barrier()

            eps = jnp.full((L,), 1e-8, jnp.float32)

            def norm_body(j, _):
                cell0 = s * PLANE + j * NORM_CHUNK
                for a in range(5):
                    pltpu.sync_copy(accs[a].at[pl.ds(cell0, NORM_CHUNK)], nbufs[a])

                def vec_body(vv, _):
                    sl = pl.ds(vv * L, L)
                    wsum = nbufs[4][sl]
                    inv = one_f / (wsum + eps)
                    for ch in range(N_CH):
                        out_bufs[ch][sl] = nbufs[ch][sl] * inv
                    return 0

                lax.fori_loop(0, NORM_CHUNK // L, vec_body, 0)
                for ch in range(N_CH):
                    pltpu.sync_copy(
                        out_bufs[ch], out.at[ch, pl.ds(gcell0 + cell0, NORM_CHUNK)]
                    )
                return 0

            lax.fori_loop(0, PLANE // NORM_CHUNK, norm_body, 0)
            plsc.subcore_barrier()
            return 0

        lax.fori_loop(0, NUM_PASSES, do_pass, 0)

    return sc_kernel


_SC_KERNEL = _make_sc_kernel()


def kernel(particle_coords, particle_values, xyz_min, xyz_max):
    scale = 127.0 / (xyz_max - xyz_min)
    off = -xyz_min * scale
    consts = jnp.repeat(jnp.concatenate([scale, off]), 16).astype(jnp.float32)
    zeros_hbm = jnp.zeros((ZROWS,), jnp.float32)
    out = _SC_KERNEL(
        particle_coords.reshape(-1), particle_values.reshape(-1), consts, zeros_hbm
    )
    return out.reshape(N_CH, D, H, W)


# speedup vs baseline: 5.0528x; 1.3542x over previous
import functools
import jax
import jax.numpy as jnp
from jax import lax
from jax.experimental import pallas as pl
from jax.experimental.pallas import tpu as pltpu
from jax.experimental.pallas import tpu_sc as plsc

D = H = W = 128
PLANE = H * W
N_PART = 2097152
N_CH = 4
PLANES_PER_REGION = 16
REGION_CELLS = PLANES_PER_REGION * PLANE
PAD_ROWS = 2048
ACC_ROWS = REGION_CELLS + PAD_ROWS
NUM_PASSES = 4
CHUNK = 128
NC, NS, L = 2, 16, 16
PER_TEC = N_PART // NS
N_CHUNKS = PER_TEC // CHUNK
ZROWS = ACC_ROWS // NS
NORM_CHUNK = 2048
CORNERS = [(di, dj, dk) for di in (0, 1) for dj in (0, 1) for dk in (0, 1)]
CBUFN = 8 * CHUNK + CHUNK


def _make_sc_kernel():
    mesh = plsc.VectorSubcoreMesh(core_axis_name="c", subcore_axis_name="s")
    scratch = []
    scratch += [pltpu.VMEM_SHARED((ACC_ROWS,), jnp.float32) for _ in range(5)]
    scratch += [
        pltpu.VMEM((CHUNK * 3,), jnp.float32),
        pltpu.VMEM((CHUNK * 4,), jnp.float32),
        pltpu.VMEM((96,), jnp.float32),
    ]
    scratch += [pltpu.VMEM((CBUFN,), jnp.int32)]
    scratch += [pltpu.VMEM((CBUFN,), jnp.float32) for _ in range(5)]
    scratch += [pltpu.VMEM((NORM_CHUNK,), jnp.float32) for _ in range(5)]
    scratch += [pltpu.VMEM((NORM_CHUNK,), jnp.float32) for _ in range(4)]
    scratch += [pltpu.SemaphoreType.DMA]

    @functools.partial(
        pl.kernel,
        mesh=mesh,
        out_type=jax.ShapeDtypeStruct((N_CH, D * H * W), jnp.float32),
        scratch_types=scratch,
        compiler_params=pltpu.CompilerParams(needs_layout_passes=False),
    )
    def sc_kernel(coords, values, consts, zeros_hbm, out, *refs):
        accs = list(refs[0:5])
        cbuf, vbuf, constv = refs[5], refs[6], refs[7]
        idxC = refs[8]
        payC = list(refs[9:14])
        nbufs = list(refs[14:19])
        out_bufs = list(refs[19:23])
        sem = refs[23]

        c = lax.axis_index("c")
        s = lax.axis_index("s")
        wid = c * NS + s

        pltpu.sync_copy(consts, constv)
        lanes = lax.iota(jnp.int32, L)

        def do_pass(p, _):
            r0_plane = (2 * p + c) * PLANES_PER_REGION
            gcell0 = r0_plane * PLANE

            for a in range(5):
                pltpu.sync_copy(zeros_hbm, accs[a].at[pl.ds(s * ZROWS, ZROWS)])
            plsc.subcore_barrier()

            sxv = constv[pl.ds(0, L)]
            syv = constv[pl.ds(16, L)]
            szv = constv[pl.ds(32, L)]
            oxv = constv[pl.ds(48, L)]
            oyv = constv[pl.ds(64, L)]
            ozv = constv[pl.ds(80, L)]
            zero_f = jnp.zeros((L,), jnp.float32)
            pad_sp = jnp.full((L,), REGION_CELLS, jnp.int32) + lanes
            hi_f = jnp.full((L,), 127.0, jnp.float32)
            hi_i = jnp.full((L,), 126, jnp.int32)
            one_f = jnp.full((L,), 1.0, jnp.float32)
            padbase = jnp.full((L,), REGION_CELLS, jnp.int32)
            r0i = jnp.full((L,), PLANES_PER_REGION, jnp.int32) * (2 * p + c)

            def chunk_body(ci, _):
                pbase = s * PER_TEC + ci * CHUNK
                pltpu.sync_copy(coords.at[pl.ds(pbase * 3, CHUNK * 3)], cbuf)
                pltpu.sync_copy(values.at[pl.ds(pbase * 4, CHUNK * 4)], vbuf)
                off = jnp.int32(0)
                for v in range(CHUNK // L):
                    q = lanes + (L * v)
                    q3 = q * 3
                    q4 = q << 2
                    x = plsc.load_gather(cbuf, [q3])
                    y = plsc.load_gather(cbuf, [q3 + 1])
                    z = plsc.load_gather(cbuf, [q3 + 2])
                    va = plsc.load_gather(vbuf, [q4])
                    vb = plsc.load_gather(vbuf, [q4 + 1])
                    vc = plsc.load_gather(vbuf, [q4 + 2])
                    vd = plsc.load_gather(vbuf, [q4 + 3])

                    fi = jnp.minimum(jnp.maximum(x * sxv + oxv, zero_f), hi_f)
                    fj = jnp.minimum(jnp.maximum(y * syv + oyv, zero_f), hi_f)
                    fk = jnp.minimum(jnp.maximum(z * szv + ozv, zero_f), hi_f)
                    bi = jnp.minimum(fi.astype(jnp.int32), hi_i)
                    bj = jnp.minimum(fj.astype(jnp.int32), hi_i)
                    bk = jnp.minimum(fk.astype(jnp.int32), hi_i)
                    f0 = fi - bi.astype(jnp.float32)
                    f1 = fj - bj.astype(jnp.float32)
                    f2 = fk - bk.astype(jnp.float32)
                    g0 = one_f - f0
                    g1 = one_f - f1
                    g2 = one_f - f2
                    reli = bi - r0i

                    for cn, (di, dj, dk) in enumerate(CORNERS):
                        wi = f0 if di else g0
                        wj = f1 if dj else g1
                        wk = f2 if dk else g2
                        wgt = wi * (wj * wk)
                        rel = reli + di
                        valid = (rel >= 0) & (rel < PLANES_PER_REGION)
                        lidx = (rel << 14) + ((bj + dj) << 7) + (bk + dk)
                        slo = pl.ds(off, L)
                        plsc.store_compressed(idxC.at[slo], lidx, mask=valid)
                        plsc.store_compressed(payC[0].at[slo], wgt * va, mask=valid)
                        plsc.store_compressed(payC[1].at[slo], wgt * vb, mask=valid)
                        plsc.store_compressed(payC[2].at[slo], wgt * vc, mask=valid)
                        plsc.store_compressed(payC[3].at[slo], wgt * vd, mask=valid)
                        plsc.store_compressed(payC[4].at[slo], wgt, mask=valid)
                        off = off + jnp.sum(valid.astype(jnp.int32))
                for kk in range(CHUNK // L):
                    slp = pl.ds(off + L * kk, L)
                    idxC[slp] = pad_sp + (L * kk)
                    for a in range(5):
                        payC[a][slp] = zero_f
                nblk = (off + 127) >> 7

                def fire(b, _):
                    slb = pl.ds(b * 128, 128)
                    cps = [
                        pltpu.async_copy(
                            payC[a].at[slb], accs[a].at[idxC.at[slb]], sem, add=True
                        )
                        for a in range(5)
                    ]
                    for cp in cps:
                        cp.wait()
                    return 0

                lax.fori_loop(0, nblk, fire, 0)
                return 0

            lax.fori_loop(0, N_CHUNKS, chunk_body, 0)
            plsc.subcore_barrier()

            eps = jnp.full((L,), 1e-8, jnp.float32)

            def norm_body(j, _):
                cell0 = s * PLANE + j * NORM_CHUNK
                for a in range(5):
                    pltpu.sync_copy(accs[a].at[pl.ds(cell0, NORM_CHUNK)], nbufs[a])

                def vec_body(vv, _):
                    sl = pl.ds(vv * L, L)
                    wsum = nbufs[4][sl]
                    inv = one_f / (wsum + eps)
                    for ch in range(N_CH):
                        out_bufs[ch][sl] = nbufs[ch][sl] * inv
                    return 0

                lax.fori_loop(0, NORM_CHUNK // L, vec_body, 0)
                for ch in range(N_CH):
                    pltpu.sync_copy(
                        out_bufs[ch], out.at[ch, pl.ds(gcell0 + cell0, NORM_CHUNK)]
                    )
                return 0

            lax.fori_loop(0, PLANE // NORM_CHUNK, norm_body, 0)
            plsc.subcore_barrier()
            return 0

        lax.fori_loop(0, NUM_PASSES, do_pass, 0)

    return sc_kernel


_SC_KERNEL = _make_sc_kernel()


def kernel(particle_coords, particle_values, xyz_min, xyz_max):
    scale = 127.0 / (xyz_max - xyz_min)
    off = -xyz_min * scale
    consts = jnp.repeat(jnp.concatenate([scale, off]), 16).astype(jnp.float32)
    zeros_hbm = jnp.zeros((ZROWS,), jnp.float32)
    out = _SC_KERNEL(
        particle_coords.reshape(-1), particle_values.reshape(-1), consts, zeros_hbm
    )
    return out.reshape(N_CH, D, H, W)
